# baseline (speedup 1.0000x reference)
import functools

import jax
import jax.numpy as jnp
from jax import lax
from jax.experimental import pallas as pl
from jax.experimental.pallas import tpu as pltpu
from jax.experimental.pallas import tpu_sc as plsc

N = 10000
E = 320000
D = 128
DOUT = 64
NG = 128

NC = 2
NS = 16
NW = NC * NS

CH = 128
CPT = 80
EPT = CPT * CH
E_PAD = NW * EPT
FAST_C = 1
CPT_F = 104
CPT_S = 2 * CPT - CPT_F
ACC_ROWS = 10240
SINK = N
ZPT = ACC_ROWS // NS

_mesh = plsc.VectorSubcoreMesh(core_axis_name="c", subcore_axis_name="s")


def _fill_f32(ref, rows, cols, val):
    nj = cols // 16

    def body(k, _):
        i = k // nj
        j = k % nj
        ref[i, pl.ds(j * 16, 16)] = jnp.full((16,), val, jnp.float32)
        return 0

    lax.fori_loop(0, rows * nj, body, 0)


@functools.partial(
    pl.kernel,
    mesh=_mesh,
    out_type=jax.ShapeDtypeStruct((NC, ACC_ROWS, D), jnp.float32),
    scratch_types=[
        pltpu.VMEM((CPT_F, CH), jnp.int32),
        pltpu.VMEM((CH,), jnp.int32),
        pltpu.VMEM((CH,), jnp.int32),
        pltpu.VMEM((CH, D), jnp.float32),
        pltpu.VMEM((CH, D), jnp.float32),
        pltpu.VMEM_SHARED((ACC_ROWS, D), jnp.float32),
        pltpu.SemaphoreType.DMA,
        pltpu.SemaphoreType.DMA,
        pltpu.SemaphoreType.DMA,
        pltpu.SemaphoreType.DMA,
        pltpu.SemaphoreType.DMA,
        pltpu.SemaphoreType.DMA,
    ],
)
def _sc_agg(y_hbm, src_hbm, dst_hbm, out_hbm, didx, sidx0, sidx1, rows0, rows1,
            acc, semg0, semg1, semi0, semi1, sems0, sems1):
    c = lax.axis_index("c")
    s = lax.axis_index("s")

    _fill_f32(rows0, CH, D, 0.0)
    for k in range(ZPT // CH):
        pltpu.sync_copy(rows0, acc.at[pl.ds(s * ZPT + k * CH, CH)])
    plsc.subcore_barrier()

    def run(cpt, rowbase):
        pltpu.sync_copy(dst_hbm.at[pl.ds(rowbase, cpt)], didx.at[pl.ds(0, cpt)])
        ebase = rowbase * CH

        def istart(g, ibuf, sem):
            pltpu.async_copy(src_hbm.at[pl.ds(ebase + g * CH, CH)], ibuf, sem)

        def iwait(g, ibuf, sem):
            pltpu.make_async_copy(src_hbm.at[pl.ds(ebase + g * CH, CH)], ibuf, sem).wait()

        def gstart(ibuf, buf, sem):
            pltpu.async_copy(y_hbm.at[ibuf], buf, sem)

        def gwait(ibuf, buf, sem):
            pltpu.make_async_copy(y_hbm.at[ibuf], buf, sem).wait()

        def sstart(g, buf, sem):
            pltpu.async_copy(buf, acc.at[didx.at[g]], sem, add=True)

        def swait(g, buf, sem):
            pltpu.make_async_copy(buf, acc.at[didx.at[g]], sem).wait()

        istart(0, sidx0, semi0)
        iwait(0, sidx0, semi0)
        gstart(sidx0, rows0, semg0)
        istart(1, sidx1, semi1)

        def eloop(p, _):
            g0 = 2 * p
            gwait(sidx0, rows0, semg0)
            iwait(g0 + 1, sidx1, semi1)

            @pl.when(g0 > 0)
            def _():
                swait(g0 - 1, rows1, sems1)

            gstart(sidx1, rows1, semg1)

            @pl.when(g0 + 2 < cpt)
            def _():
                istart(g0 + 2, sidx0, semi0)

            sstart(g0, rows0, sems0)
            gwait(sidx1, rows1, semg1)

            @pl.when(g0 + 2 < cpt)
            def _():
                iwait(g0 + 2, sidx0, semi0)
                swait(g0, rows0, sems0)
                gstart(sidx0, rows0, semg0)

            @pl.when(g0 + 3 < cpt)
            def _():
                istart(g0 + 3, sidx1, semi1)

            sstart(g0 + 1, rows1, sems1)
            return 0

        lax.fori_loop(0, cpt // 2, eloop, 0)
        swait(cpt - 2, rows0, sems0)
        swait(cpt - 1, rows1, sems1)

    @pl.when(c == FAST_C)
    def _():
        run(CPT_F, s * CPT_F)

    @pl.when(c != FAST_C)
    def _():
        run(CPT_S, NS * CPT_F + s * CPT_S)

    plsc.subcore_barrier()

    def wloop(k, _):
        r0 = s * ZPT + k * CH
        pltpu.sync_copy(acc.at[pl.ds(r0, CH)], rows0)
        pltpu.sync_copy(rows0, out_hbm.at[c, pl.ds(r0, CH)])
        return 0

    lax.fori_loop(0, ZPT // CH, wloop, 0)


@functools.partial(
    pl.kernel,
    mesh=_mesh,
    out_type=jax.ShapeDtypeStruct((NC, ACC_ROWS, D), jnp.float32),
    scratch_types=[
        pltpu.VMEM((CPT, CH), jnp.int32),
        pltpu.VMEM((CH, D), jnp.float32),
        pltpu.VMEM_SHARED((ACC_ROWS, D), jnp.float32),
        pltpu.SemaphoreType.DMA,
    ],
)
def _sc_count(dst_hbm, out_hbm, didx, buf, acc, sem):
    c = lax.axis_index("c")
    s = lax.axis_index("s")
    wid = s * NC + c

    pltpu.sync_copy(dst_hbm.at[pl.ds(wid * CPT, CPT)], didx)
    _fill_f32(buf, CH, D, 0.0)
    for k in range(ZPT // CH):
        pltpu.sync_copy(buf, acc.at[pl.ds(s * ZPT + k * CH, CH)])
    _fill_f32(buf, CH, D, 1.0)
    plsc.subcore_barrier()

    def eloop(g, _):
        pltpu.sync_copy(buf, acc.at[didx.at[g]], add=True)
        return 0

    lax.fori_loop(0, CPT, eloop, 0)
    plsc.subcore_barrier()

    def wloop(k, _):
        r0 = s * ZPT + k * CH
        pltpu.sync_copy(acc.at[pl.ds(r0, CH)], buf)
        pltpu.sync_copy(buf, out_hbm.at[c, pl.ds(r0, CH)])
        return 0

    lax.fori_loop(0, ZPT // CH, wloop, 0)


BLK = 1000


def _tc_mm1_body(x_ref, w_ref, b_ref, s_ref, y_ref):
    z = jnp.dot(x_ref[...], w_ref[...], preferred_element_type=jnp.float32)
    s_ref[...] = z[:, :D] + b_ref[...]
    y_ref[...] = z[:, D:]


def _tc_mm1(x, wcat, b):
    return pl.pallas_call(
        _tc_mm1_body,
        grid=(N // BLK,),
        in_specs=[
            pl.BlockSpec((BLK, D), lambda i: (i, 0)),
            pl.BlockSpec((D, 2 * D), lambda i: (0, 0)),
            pl.BlockSpec((1, D), lambda i: (0, 0)),
        ],
        out_specs=[
            pl.BlockSpec((BLK, D), lambda i: (i, 0)),
            pl.BlockSpec((BLK, D), lambda i: (i, 0)),
        ],
        out_shape=[
            jax.ShapeDtypeStruct((N, D), jnp.float32),
            jax.ShapeDtypeStruct((N, D), jnp.float32),
        ],
    )(x, wcat, b)


def _relu_h(sp_ref, a_ref, c_ref):
    cnt = c_ref[0, :, 0:1] + c_ref[1, :, 0:1]
    inv = 1.0 / jnp.maximum(cnt, 1.0)
    return jnp.maximum(sp_ref[...] + (a_ref[0] + a_ref[1]) * inv, 0.0)


def _tc_mml_body(sp_ref, a_ref, c_ref, w_ref, b_ref, s_ref, y_ref):
    h = _relu_h(sp_ref, a_ref, c_ref)
    z = jnp.dot(h, w_ref[...], preferred_element_type=jnp.float32)
    s_ref[...] = z[:, :D] + b_ref[...]
    y_ref[...] = z[:, D:]


def _tc_mml(sp, agg, cnt, wcat, b):
    return pl.pallas_call(
        _tc_mml_body,
        grid=(N // BLK,),
        in_specs=[
            pl.BlockSpec((BLK, D), lambda i: (i, 0)),
            pl.BlockSpec((NC, BLK, D), lambda i: (0, i, 0)),
            pl.BlockSpec((NC, BLK, D), lambda i: (0, i, 0)),
            pl.BlockSpec((D, 2 * D), lambda i: (0, 0)),
            pl.BlockSpec((1, D), lambda i: (0, 0)),
        ],
        out_specs=[
            pl.BlockSpec((BLK, D), lambda i: (i, 0)),
            pl.BlockSpec((BLK, D), lambda i: (i, 0)),
        ],
        out_shape=[
            jax.ShapeDtypeStruct((N, D), jnp.float32),
            jax.ShapeDtypeStruct((N, D), jnp.float32),
        ],
    )(sp, agg, cnt, wcat, b)


def _tc_pool_body(sp_ref, a_ref, c_ref, batch_ref, wl_ref, bl_ref, out_ref, gacc, cacc):
    i = pl.program_id(0)

    @pl.when(i == 0)
    def _():
        gacc[...] = jnp.zeros_like(gacc)
        cacc[...] = jnp.zeros_like(cacc)

    h = _relu_h(sp_ref, a_ref, c_ref)
    b = batch_ref[0, 0, :]
    mask = (b[None, :] == lax.broadcasted_iota(jnp.int32, (NG, BLK), 0)).astype(jnp.float32)
    gacc[...] += jnp.dot(mask, h, preferred_element_type=jnp.float32)
    cacc[...] += jnp.broadcast_to(jnp.sum(mask, axis=1, keepdims=True), (NG, NG))

    @pl.when(i == pl.num_programs(0) - 1)
    def _():
        g = gacc[...] / jnp.maximum(cacc[:, 0:1], 1.0)
        out_ref[...] = jnp.dot(g, wl_ref[...], preferred_element_type=jnp.float32) + bl_ref[...]


def _tc_pool(sp, agg, cnt, batch2d, wl, bl):
    return pl.pallas_call(
        _tc_pool_body,
        grid=(N // BLK,),
        in_specs=[
            pl.BlockSpec((BLK, D), lambda i: (i, 0)),
            pl.BlockSpec((NC, BLK, D), lambda i: (0, i, 0)),
            pl.BlockSpec((NC, BLK, D), lambda i: (0, i, 0)),
            pl.BlockSpec((1, 1, BLK), lambda i: (i, 0, 0)),
            pl.BlockSpec((D, DOUT), lambda i: (0, 0)),
            pl.BlockSpec((1, DOUT), lambda i: (0, 0)),
        ],
        out_specs=pl.BlockSpec((NG, DOUT), lambda i: (0, 0)),
        out_shape=jax.ShapeDtypeStruct((NG, DOUT), jnp.float32),
        scratch_shapes=[
            pltpu.VMEM((NG, NG), jnp.float32),
            pltpu.VMEM((NG, NG), jnp.float32),
        ],
    )(sp, agg, cnt, batch2d, wl, bl)


def kernel(x, edge_index, batch, W1_self, W1_neigh, b1, W2_self, W2_neigh, b2,
           W3_self, W3_neigh, b3, W_lin, b_lin):
    src = edge_index[0].astype(jnp.int32)
    dst = edge_index[1].astype(jnp.int32)
    pad = E_PAD - E
    src = jnp.concatenate([src, jnp.zeros((pad,), jnp.int32)])
    dst = jnp.concatenate([dst, jnp.full((pad,), SINK, jnp.int32)]).reshape(NW * CPT, CH)
    batch2d = batch.astype(jnp.int32).reshape(N // BLK, 1, BLK)

    w1 = jnp.concatenate([W1_self, W1_neigh], axis=1)
    w2 = jnp.concatenate([W2_self, W2_neigh], axis=1)
    w3 = jnp.concatenate([W3_self, W3_neigh], axis=1)

    cnt = _sc_count(dst)
    s, y = _tc_mm1(x, w1, b1.reshape(1, D))
    agg = _sc_agg(y, src, dst)
    s, y = _tc_mml(s, agg, cnt, w2, b2.reshape(1, D))
    agg = _sc_agg(y, src, dst)
    s, y = _tc_mml(s, agg, cnt, w3, b3.reshape(1, D))
    agg = _sc_agg(y, src, dst)
    return _tc_pool(s, agg, cnt, batch2d, W_lin, b_lin.reshape(1, DOUT))


# speedup vs baseline: 1.0221x; 1.0221x over previous
import functools

import jax
import jax.numpy as jnp
from jax import lax
from jax.experimental import pallas as pl
from jax.experimental.pallas import tpu as pltpu
from jax.experimental.pallas import tpu_sc as plsc

N = 10000
E = 320000
D = 128
DOUT = 64
NG = 128

NC = 2
NS = 16
NW = NC * NS

CH = 128
CPT = 80
EPT = CPT * CH
E_PAD = NW * EPT
FAST_C = 1
CPT_F = 120
CPT_S = 2 * CPT - CPT_F
ACC_ROWS = 10240
SINK = N
ZPT = ACC_ROWS // NS

_mesh = plsc.VectorSubcoreMesh(core_axis_name="c", subcore_axis_name="s")


def _fill_f32(ref, rows, cols, val):
    nj = cols // 16

    def body(k, _):
        i = k // nj
        j = k % nj
        ref[i, pl.ds(j * 16, 16)] = jnp.full((16,), val, jnp.float32)
        return 0

    lax.fori_loop(0, rows * nj, body, 0)


@functools.partial(
    pl.kernel,
    mesh=_mesh,
    out_type=jax.ShapeDtypeStruct((NC, ACC_ROWS, D), jnp.float32),
    scratch_types=[
        pltpu.VMEM((CPT_F, CH), jnp.int32),
        pltpu.VMEM((CH,), jnp.int32),
        pltpu.VMEM((CH,), jnp.int32),
        pltpu.VMEM((CH, D), jnp.float32),
        pltpu.VMEM((CH, D), jnp.float32),
        pltpu.VMEM_SHARED((ACC_ROWS, D), jnp.float32),
        pltpu.SemaphoreType.DMA,
        pltpu.SemaphoreType.DMA,
        pltpu.SemaphoreType.DMA,
        pltpu.SemaphoreType.DMA,
        pltpu.SemaphoreType.DMA,
        pltpu.SemaphoreType.DMA,
    ],
)
def _sc_agg(y_hbm, src_hbm, dst_hbm, out_hbm, didx, sidx0, sidx1, rows0, rows1,
            acc, semg0, semg1, semi0, semi1, sems0, sems1):
    c = lax.axis_index("c")
    s = lax.axis_index("s")

    _fill_f32(rows0, CH, D, 0.0)
    for k in range(ZPT // CH):
        pltpu.sync_copy(rows0, acc.at[pl.ds(s * ZPT + k * CH, CH)])
    plsc.subcore_barrier()

    def run(cpt, rowbase):
        pltpu.sync_copy(dst_hbm.at[pl.ds(rowbase, cpt)], didx.at[pl.ds(0, cpt)])
        ebase = rowbase * CH

        def istart(g, ibuf, sem):
            pltpu.async_copy(src_hbm.at[pl.ds(ebase + g * CH, CH)], ibuf, sem)

        def iwait(g, ibuf, sem):
            pltpu.make_async_copy(src_hbm.at[pl.ds(ebase + g * CH, CH)], ibuf, sem).wait()

        def gstart(ibuf, buf, sem):
            pltpu.async_copy(y_hbm.at[ibuf], buf, sem)

        def gwait(ibuf, buf, sem):
            pltpu.make_async_copy(y_hbm.at[ibuf], buf, sem).wait()

        def sstart(g, buf, sem):
            pltpu.async_copy(buf, acc.at[didx.at[g]], sem, add=True)

        def swait(g, buf, sem):
            pltpu.make_async_copy(buf, acc.at[didx.at[g]], sem).wait()

        istart(0, sidx0, semi0)
        iwait(0, sidx0, semi0)
        gstart(sidx0, rows0, semg0)
        istart(1, sidx1, semi1)

        def eloop(p, _):
            g0 = 2 * p
            gwait(sidx0, rows0, semg0)
            iwait(g0 + 1, sidx1, semi1)

            @pl.when(g0 > 0)
            def _():
                swait(g0 - 1, rows1, sems1)

            gstart(sidx1, rows1, semg1)

            @pl.when(g0 + 2 < cpt)
            def _():
                istart(g0 + 2, sidx0, semi0)

            sstart(g0, rows0, sems0)
            gwait(sidx1, rows1, semg1)

            @pl.when(g0 + 2 < cpt)
            def _():
                iwait(g0 + 2, sidx0, semi0)
                swait(g0, rows0, sems0)
                gstart(sidx0, rows0, semg0)

            @pl.when(g0 + 3 < cpt)
            def _():
                istart(g0 + 3, sidx1, semi1)

            sstart(g0 + 1, rows1, sems1)
            return 0

        lax.fori_loop(0, cpt // 2, eloop, 0)
        swait(cpt - 2, rows0, sems0)
        swait(cpt - 1, rows1, sems1)

    @pl.when(c == FAST_C)
    def _():
        run(CPT_F, s * CPT_F)

    @pl.when(c != FAST_C)
    def _():
        run(CPT_S, NS * CPT_F + s * CPT_S)

    plsc.subcore_barrier()

    def wloop(k, _):
        r0 = s * ZPT + k * CH
        pltpu.sync_copy(acc.at[pl.ds(r0, CH)], rows0)
        pltpu.sync_copy(rows0, out_hbm.at[c, pl.ds(r0, CH)])
        return 0

    lax.fori_loop(0, ZPT // CH, wloop, 0)


@functools.partial(
    pl.kernel,
    mesh=_mesh,
    out_type=jax.ShapeDtypeStruct((NC, ACC_ROWS, D), jnp.float32),
    scratch_types=[
        pltpu.VMEM((CPT, CH), jnp.int32),
        pltpu.VMEM((CH, D), jnp.float32),
        pltpu.VMEM_SHARED((ACC_ROWS, D), jnp.float32),
        pltpu.SemaphoreType.DMA,
    ],
)
def _sc_count(dst_hbm, out_hbm, didx, buf, acc, sem):
    c = lax.axis_index("c")
    s = lax.axis_index("s")
    wid = s * NC + c

    pltpu.sync_copy(dst_hbm.at[pl.ds(wid * CPT, CPT)], didx)
    _fill_f32(buf, CH, D, 0.0)
    for k in range(ZPT // CH):
        pltpu.sync_copy(buf, acc.at[pl.ds(s * ZPT + k * CH, CH)])
    _fill_f32(buf, CH, D, 1.0)
    plsc.subcore_barrier()

    def eloop(g, _):
        pltpu.sync_copy(buf, acc.at[didx.at[g]], add=True)
        return 0

    lax.fori_loop(0, CPT, eloop, 0)
    plsc.subcore_barrier()

    def wloop(k, _):
        r0 = s * ZPT + k * CH
        pltpu.sync_copy(acc.at[pl.ds(r0, CH)], buf)
        pltpu.sync_copy(buf, out_hbm.at[c, pl.ds(r0, CH)])
        return 0

    lax.fori_loop(0, ZPT // CH, wloop, 0)


BLK = 1000


def _tc_mm1_body(x_ref, w_ref, b_ref, s_ref, y_ref):
    z = jnp.dot(x_ref[...], w_ref[...], preferred_element_type=jnp.float32)
    s_ref[...] = z[:, :D] + b_ref[...]
    y_ref[...] = z[:, D:]


def _tc_mm1(x, wcat, b):
    return pl.pallas_call(
        _tc_mm1_body,
        grid=(N // BLK,),
        in_specs=[
            pl.BlockSpec((BLK, D), lambda i: (i, 0)),
            pl.BlockSpec((D, 2 * D), lambda i: (0, 0)),
            pl.BlockSpec((1, D), lambda i: (0, 0)),
        ],
        out_specs=[
            pl.BlockSpec((BLK, D), lambda i: (i, 0)),
            pl.BlockSpec((BLK, D), lambda i: (i, 0)),
        ],
        out_shape=[
            jax.ShapeDtypeStruct((N, D), jnp.float32),
            jax.ShapeDtypeStruct((N, D), jnp.float32),
        ],
    )(x, wcat, b)


def _relu_h(sp_ref, a_ref, c_ref):
    cnt = c_ref[0, :, 0:1] + c_ref[1, :, 0:1]
    inv = 1.0 / jnp.maximum(cnt, 1.0)
    return jnp.maximum(sp_ref[...] + (a_ref[0] + a_ref[1]) * inv, 0.0)


def _tc_mml_body(sp_ref, a_ref, c_ref, w_ref, b_ref, s_ref, y_ref):
    h = _relu_h(sp_ref, a_ref, c_ref)
    z = jnp.dot(h, w_ref[...], preferred_element_type=jnp.float32)
    s_ref[...] = z[:, :D] + b_ref[...]
    y_ref[...] = z[:, D:]


def _tc_mml(sp, agg, cnt, wcat, b):
    return pl.pallas_call(
        _tc_mml_body,
        grid=(N // BLK,),
        in_specs=[
            pl.BlockSpec((BLK, D), lambda i: (i, 0)),
            pl.BlockSpec((NC, BLK, D), lambda i: (0, i, 0)),
            pl.BlockSpec((NC, BLK, D), lambda i: (0, i, 0)),
            pl.BlockSpec((D, 2 * D), lambda i: (0, 0)),
            pl.BlockSpec((1, D), lambda i: (0, 0)),
        ],
        out_specs=[
            pl.BlockSpec((BLK, D), lambda i: (i, 0)),
            pl.BlockSpec((BLK, D), lambda i: (i, 0)),
        ],
        out_shape=[
            jax.ShapeDtypeStruct((N, D), jnp.float32),
            jax.ShapeDtypeStruct((N, D), jnp.float32),
        ],
    )(sp, agg, cnt, wcat, b)


def _tc_pool_body(sp_ref, a_ref, c_ref, batch_ref, wl_ref, bl_ref, out_ref, gacc, cacc):
    i = pl.program_id(0)

    @pl.when(i == 0)
    def _():
        gacc[...] = jnp.zeros_like(gacc)
        cacc[...] = jnp.zeros_like(cacc)

    h = _relu_h(sp_ref, a_ref, c_ref)
    b = batch_ref[0, 0, :]
    mask = (b[None, :] == lax.broadcasted_iota(jnp.int32, (NG, BLK), 0)).astype(jnp.float32)
    gacc[...] += jnp.dot(mask, h, preferred_element_type=jnp.float32)
    cacc[...] += jnp.broadcast_to(jnp.sum(mask, axis=1, keepdims=True), (NG, NG))

    @pl.when(i == pl.num_programs(0) - 1)
    def _():
        g = gacc[...] / jnp.maximum(cacc[:, 0:1], 1.0)
        out_ref[...] = jnp.dot(g, wl_ref[...], preferred_element_type=jnp.float32) + bl_ref[...]


def _tc_pool(sp, agg, cnt, batch2d, wl, bl):
    return pl.pallas_call(
        _tc_pool_body,
        grid=(N // BLK,),
        in_specs=[
            pl.BlockSpec((BLK, D), lambda i: (i, 0)),
            pl.BlockSpec((NC, BLK, D), lambda i: (0, i, 0)),
            pl.BlockSpec((NC, BLK, D), lambda i: (0, i, 0)),
            pl.BlockSpec((1, 1, BLK), lambda i: (i, 0, 0)),
            pl.BlockSpec((D, DOUT), lambda i: (0, 0)),
            pl.BlockSpec((1, DOUT), lambda i: (0, 0)),
        ],
        out_specs=pl.BlockSpec((NG, DOUT), lambda i: (0, 0)),
        out_shape=jax.ShapeDtypeStruct((NG, DOUT), jnp.float32),
        scratch_shapes=[
            pltpu.VMEM((NG, NG), jnp.float32),
            pltpu.VMEM((NG, NG), jnp.float32),
        ],
    )(sp, agg, cnt, batch2d, wl, bl)


def kernel(x, edge_index, batch, W1_self, W1_neigh, b1, W2_self, W2_neigh, b2,
           W3_self, W3_neigh, b3, W_lin, b_lin):
    src = edge_index[0].astype(jnp.int32)
    dst = edge_index[1].astype(jnp.int32)
    pad = E_PAD - E
    src = jnp.concatenate([src, jnp.zeros((pad,), jnp.int32)])
    dst = jnp.concatenate([dst, jnp.full((pad,), SINK, jnp.int32)]).reshape(NW * CPT, CH)
    batch2d = batch.astype(jnp.int32).reshape(N // BLK, 1, BLK)

    w1 = jnp.concatenate([W1_self, W1_neigh], axis=1)
    w2 = jnp.concatenate([W2_self, W2_neigh], axis=1)
    w3 = jnp.concatenate([W3_self, W3_neigh], axis=1)

    cnt = _sc_count(dst)
    s, y = _tc_mm1(x, w1, b1.reshape(1, D))
    agg = _sc_agg(y, src, dst)
    s, y = _tc_mml(s, agg, cnt, w2, b2.reshape(1, D))
    agg = _sc_agg(y, src, dst)
    s, y = _tc_mml(s, agg, cnt, w3, b3.reshape(1, D))
    agg = _sc_agg(y, src, dst)
    return _tc_pool(s, agg, cnt, batch2d, W_lin, b_lin.reshape(1, DOUT))


# speedup vs baseline: 1.0319x; 1.0096x over previous
import functools

import jax
import jax.numpy as jnp
from jax import lax
from jax.experimental import pallas as pl
from jax.experimental.pallas import tpu as pltpu
from jax.experimental.pallas import tpu_sc as plsc

N = 10000
E = 320000
D = 128
DOUT = 64
NG = 128

NC = 2
NS = 16
NW = NC * NS

CH = 128
CPT = 80
EPT = CPT * CH
E_PAD = NW * EPT
FAST_C = 1
CPT_F = 128
CPT_S = 2 * CPT - CPT_F
ACC_ROWS = 10112
SINK = N
ZPT = ACC_ROWS // NS
ZCHUNKS = ((CH, 0), (CH, CH), (CH, 2 * CH), (CH, 3 * CH), (ZPT - 4 * CH, 4 * CH))

_mesh = plsc.VectorSubcoreMesh(core_axis_name="c", subcore_axis_name="s")


def _fill_f32(ref, rows, cols, val):
    nj = cols // 16

    def body(k, _):
        i = k // nj
        j = k % nj
        ref[i, pl.ds(j * 16, 16)] = jnp.full((16,), val, jnp.float32)
        return 0

    lax.fori_loop(0, rows * nj, body, 0)


@functools.partial(
    pl.kernel,
    mesh=_mesh,
    out_type=jax.ShapeDtypeStruct((NC, ACC_ROWS, D), jnp.float32),
    scratch_types=[
        pltpu.VMEM((CPT_F, CH), jnp.int32),
        pltpu.VMEM((CH,), jnp.int32),
        pltpu.VMEM((CH,), jnp.int32),
        pltpu.VMEM((CH, D), jnp.float32),
        pltpu.VMEM((CH, D), jnp.float32),
        pltpu.VMEM_SHARED((ACC_ROWS, D), jnp.float32),
        pltpu.SemaphoreType.DMA,
        pltpu.SemaphoreType.DMA,
        pltpu.SemaphoreType.DMA,
        pltpu.SemaphoreType.DMA,
        pltpu.SemaphoreType.DMA,
        pltpu.SemaphoreType.DMA,
    ],
)
def _sc_agg(y_hbm, src_hbm, dst_hbm, out_hbm, didx, sidx0, sidx1, rows0, rows1,
            acc, semg0, semg1, semi0, semi1, sems0, sems1):
    c = lax.axis_index("c")
    s = lax.axis_index("s")

    _fill_f32(rows0, CH, D, 0.0)
    for sz, off in ZCHUNKS:
        pltpu.sync_copy(rows0.at[pl.ds(0, sz)], acc.at[pl.ds(s * ZPT + off, sz)])
    plsc.subcore_barrier()

    def run(cpt, rowbase):
        pltpu.sync_copy(dst_hbm.at[pl.ds(rowbase, cpt)], didx.at[pl.ds(0, cpt)])
        ebase = rowbase * CH

        def istart(g, ibuf, sem):
            pltpu.async_copy(src_hbm.at[pl.ds(ebase + g * CH, CH)], ibuf, sem)

        def iwait(g, ibuf, sem):
            pltpu.make_async_copy(src_hbm.at[pl.ds(ebase + g * CH, CH)], ibuf, sem).wait()

        def gstart(ibuf, buf, sem):
            pltpu.async_copy(y_hbm.at[ibuf], buf, sem)

        def gwait(ibuf, buf, sem):
            pltpu.make_async_copy(y_hbm.at[ibuf], buf, sem).wait()

        def sstart(g, buf, sem):
            pltpu.async_copy(buf, acc.at[didx.at[g]], sem, add=True)

        def swait(g, buf, sem):
            pltpu.make_async_copy(buf, acc.at[didx.at[g]], sem).wait()

        istart(0, sidx0, semi0)
        iwait(0, sidx0, semi0)
        gstart(sidx0, rows0, semg0)
        istart(1, sidx1, semi1)

        def eloop(p, _):
            g0 = 2 * p
            gwait(sidx0, rows0, semg0)
            iwait(g0 + 1, sidx1, semi1)

            @pl.when(g0 > 0)
            def _():
                swait(g0 - 1, rows1, sems1)

            gstart(sidx1, rows1, semg1)

            @pl.when(g0 + 2 < cpt)
            def _():
                istart(g0 + 2, sidx0, semi0)

            sstart(g0, rows0, sems0)
            gwait(sidx1, rows1, semg1)

            @pl.when(g0 + 2 < cpt)
            def _():
                iwait(g0 + 2, sidx0, semi0)
                swait(g0, rows0, sems0)
                gstart(sidx0, rows0, semg0)

            @pl.when(g0 + 3 < cpt)
            def _():
                istart(g0 + 3, sidx1, semi1)

            sstart(g0 + 1, rows1, sems1)
            return 0

        lax.fori_loop(0, cpt // 2, eloop, 0)
        swait(cpt - 2, rows0, sems0)
        swait(cpt - 1, rows1, sems1)

    @pl.when(c == FAST_C)
    def _():
        run(CPT_F, s * CPT_F)

    @pl.when(c != FAST_C)
    def _():
        run(CPT_S, NS * CPT_F + s * CPT_S)

    plsc.subcore_barrier()

    for sz, off in ZCHUNKS:
        r0 = s * ZPT + off
        pltpu.sync_copy(acc.at[pl.ds(r0, sz)], rows0.at[pl.ds(0, sz)])
        pltpu.sync_copy(rows0.at[pl.ds(0, sz)], out_hbm.at[c, pl.ds(r0, sz)])


@functools.partial(
    pl.kernel,
    mesh=_mesh,
    out_type=jax.ShapeDtypeStruct((NC, ACC_ROWS, D), jnp.float32),
    scratch_types=[
        pltpu.VMEM((CPT, CH), jnp.int32),
        pltpu.VMEM((CH, D), jnp.float32),
        pltpu.VMEM_SHARED((ACC_ROWS, D), jnp.float32),
        pltpu.SemaphoreType.DMA,
    ],
)
def _sc_count(dst_hbm, out_hbm, didx, buf, acc, sem):
    c = lax.axis_index("c")
    s = lax.axis_index("s")
    wid = s * NC + c

    pltpu.sync_copy(dst_hbm.at[pl.ds(wid * CPT, CPT)], didx)
    _fill_f32(buf, CH, D, 0.0)
    for sz, off in ZCHUNKS:
        pltpu.sync_copy(buf.at[pl.ds(0, sz)], acc.at[pl.ds(s * ZPT + off, sz)])
    _fill_f32(buf, CH, D, 1.0)
    plsc.subcore_barrier()

    def eloop(g, _):
        pltpu.sync_copy(buf, acc.at[didx.at[g]], add=True)
        return 0

    lax.fori_loop(0, CPT, eloop, 0)
    plsc.subcore_barrier()

    for sz, off in ZCHUNKS:
        r0 = s * ZPT + off
        pltpu.sync_copy(acc.at[pl.ds(r0, sz)], buf.at[pl.ds(0, sz)])
        pltpu.sync_copy(buf.at[pl.ds(0, sz)], out_hbm.at[c, pl.ds(r0, sz)])


BLK = 1000


def _tc_mm1_body(x_ref, w_ref, b_ref, s_ref, y_ref):
    z = jnp.dot(x_ref[...], w_ref[...], preferred_element_type=jnp.float32)
    s_ref[...] = z[:, :D] + b_ref[...]
    y_ref[...] = z[:, D:]


def _tc_mm1(x, wcat, b):
    return pl.pallas_call(
        _tc_mm1_body,
        grid=(N // BLK,),
        in_specs=[
            pl.BlockSpec((BLK, D), lambda i: (i, 0)),
            pl.BlockSpec((D, 2 * D), lambda i: (0, 0)),
            pl.BlockSpec((1, D), lambda i: (0, 0)),
        ],
        out_specs=[
            pl.BlockSpec((BLK, D), lambda i: (i, 0)),
            pl.BlockSpec((BLK, D), lambda i: (i, 0)),
        ],
        out_shape=[
            jax.ShapeDtypeStruct((N, D), jnp.float32),
            jax.ShapeDtypeStruct((N, D), jnp.float32),
        ],
    )(x, wcat, b)


def _relu_h(sp_ref, a_ref, c_ref):
    cnt = c_ref[0, :, 0:1] + c_ref[1, :, 0:1]
    inv = 1.0 / jnp.maximum(cnt, 1.0)
    return jnp.maximum(sp_ref[...] + (a_ref[0] + a_ref[1]) * inv, 0.0)


def _tc_mml_body(sp_ref, a_ref, c_ref, w_ref, b_ref, s_ref, y_ref):
    h = _relu_h(sp_ref, a_ref, c_ref)
    z = jnp.dot(h, w_ref[...], preferred_element_type=jnp.float32)
    s_ref[...] = z[:, :D] + b_ref[...]
    y_ref[...] = z[:, D:]


def _tc_mml(sp, agg, cnt, wcat, b):
    return pl.pallas_call(
        _tc_mml_body,
        grid=(N // BLK,),
        in_specs=[
            pl.BlockSpec((BLK, D), lambda i: (i, 0)),
            pl.BlockSpec((NC, BLK, D), lambda i: (0, i, 0)),
            pl.BlockSpec((NC, BLK, D), lambda i: (0, i, 0)),
            pl.BlockSpec((D, 2 * D), lambda i: (0, 0)),
            pl.BlockSpec((1, D), lambda i: (0, 0)),
        ],
        out_specs=[
            pl.BlockSpec((BLK, D), lambda i: (i, 0)),
            pl.BlockSpec((BLK, D), lambda i: (i, 0)),
        ],
        out_shape=[
            jax.ShapeDtypeStruct((N, D), jnp.float32),
            jax.ShapeDtypeStruct((N, D), jnp.float32),
        ],
    )(sp, agg, cnt, wcat, b)


def _tc_pool_body(sp_ref, a_ref, c_ref, batch_ref, wl_ref, bl_ref, out_ref, gacc, cacc):
    i = pl.program_id(0)

    @pl.when(i == 0)
    def _():
        gacc[...] = jnp.zeros_like(gacc)
        cacc[...] = jnp.zeros_like(cacc)

    h = _relu_h(sp_ref, a_ref, c_ref)
    b = batch_ref[0, 0, :]
    mask = (b[None, :] == lax.broadcasted_iota(jnp.int32, (NG, BLK), 0)).astype(jnp.float32)
    gacc[...] += jnp.dot(mask, h, preferred_element_type=jnp.float32)
    cacc[...] += jnp.broadcast_to(jnp.sum(mask, axis=1, keepdims=True), (NG, NG))

    @pl.when(i == pl.num_programs(0) - 1)
    def _():
        g = gacc[...] / jnp.maximum(cacc[:, 0:1], 1.0)
        out_ref[...] = jnp.dot(g, wl_ref[...], preferred_element_type=jnp.float32) + bl_ref[...]


def _tc_pool(sp, agg, cnt, batch2d, wl, bl):
    return pl.pallas_call(
        _tc_pool_body,
        grid=(N // BLK,),
        in_specs=[
            pl.BlockSpec((BLK, D), lambda i: (i, 0)),
            pl.BlockSpec((NC, BLK, D), lambda i: (0, i, 0)),
            pl.BlockSpec((NC, BLK, D), lambda i: (0, i, 0)),
            pl.BlockSpec((1, 1, BLK), lambda i: (i, 0, 0)),
            pl.BlockSpec((D, DOUT), lambda i: (0, 0)),
            pl.BlockSpec((1, DOUT), lambda i: (0, 0)),
        ],
        out_specs=pl.BlockSpec((NG, DOUT), lambda i: (0, 0)),
        out_shape=jax.ShapeDtypeStruct((NG, DOUT), jnp.float32),
        scratch_shapes=[
            pltpu.VMEM((NG, NG), jnp.float32),
            pltpu.VMEM((NG, NG), jnp.float32),
        ],
    )(sp, agg, cnt, batch2d, wl, bl)


def kernel(x, edge_index, batch, W1_self, W1_neigh, b1, W2_self, W2_neigh, b2,
           W3_self, W3_neigh, b3, W_lin, b_lin):
    src = edge_index[0].astype(jnp.int32)
    dst = edge_index[1].astype(jnp.int32)
    pad = E_PAD - E
    src = jnp.concatenate([src, jnp.zeros((pad,), jnp.int32)])
    dst = jnp.concatenate([dst, jnp.full((pad,), SINK, jnp.int32)]).reshape(NW * CPT, CH)
    batch2d = batch.astype(jnp.int32).reshape(N // BLK, 1, BLK)

    w1 = jnp.concatenate([W1_self, W1_neigh], axis=1)
    w2 = jnp.concatenate([W2_self, W2_neigh], axis=1)
    w3 = jnp.concatenate([W3_self, W3_neigh], axis=1)

    cnt = _sc_count(dst)
    s, y = _tc_mm1(x, w1, b1.reshape(1, D))
    agg = _sc_agg(y, src, dst)
    s, y = _tc_mml(s, agg, cnt, w2, b2.reshape(1, D))
    agg = _sc_agg(y, src, dst)
    s, y = _tc_mml(s, agg, cnt, w3, b3.reshape(1, D))
    agg = _sc_agg(y, src, dst)
    return _tc_pool(s, agg, cnt, batch2d, W_lin, b_lin.reshape(1, DOUT))
